# Initial kernel scaffold; baseline (speedup 1.0000x reference)
import jax
import jax.numpy as jnp
from jax.experimental import pallas as pl


def kernel(x0, x1, length_0, length_1, embedding, W, b):
    raise NotImplementedError("write your pallas kernel here")


# speedup vs baseline: 1.4679x; 1.4679x over previous
import functools

import jax
import jax.numpy as jnp
from jax import lax
from jax.experimental import pallas as pl
from jax.experimental.pallas import tpu as pltpu
from jax.experimental.pallas import tpu_sc as plsc

DIM = 64
L = 50
EX_PER_CHUNK = 2
CHUNK = EX_PER_CHUNK * L
NC, NS = 2, 16
NW = NC * NS
NVREG = DIM // 16


def _make_pool(total_chunks: int):
    tpw = total_chunks // NW
    epw = tpw * EX_PER_CHUNK

    mesh = plsc.VectorSubcoreMesh(core_axis_name="c", subcore_axis_name="s")

    @functools.partial(
        pl.kernel,
        out_type=jax.ShapeDtypeStruct((NW, epw * DIM), jnp.float32),
        mesh=mesh,
        scratch_types=[
            pltpu.VMEM((tpw, CHUNK), jnp.int32),
            pltpu.VMEM((CHUNK, DIM), jnp.float32),
            pltpu.VMEM((CHUNK, DIM), jnp.float32),
            pltpu.VMEM((epw * DIM,), jnp.float32),
            pltpu.SemaphoreType.DMA,
            pltpu.SemaphoreType.DMA,
        ],
        compiler_params=pltpu.CompilerParams(use_tc_tiling_on_sc=False),
    )
    def pool(idx_hbm, emb_hbm, out_hbm, idx_v, rows0, rows1, out_v, sem0, sem1):
        wid = lax.axis_index("c") * NS + lax.axis_index("s")
        pltpu.sync_copy(idx_hbm.at[pl.ds(wid * tpw, tpw)], idx_v)

        def reduce_chunk(rows, local_base):
            for e in range(EX_PER_CHUNK):
                acc = [rows[e * L, pl.ds(c * 16, 16)] for c in range(NVREG)]
                for r in range(1, L):
                    for c in range(NVREG):
                        acc[c] = acc[c] + rows[e * L + r, pl.ds(c * 16, 16)]
                off = (local_base + e) * DIM
                for c in range(NVREG):
                    out_v[pl.ds(off + c * 16, 16)] = acc[c]

        pltpu.async_copy(emb_hbm.at[idx_v.at[0]], rows0, sem0)
        pltpu.async_copy(emb_hbm.at[idx_v.at[1]], rows1, sem1)

        @pl.loop(0, tpw, step=2)
        def _(i):
            pltpu.make_async_copy(emb_hbm.at[idx_v.at[i]], rows0, sem0).wait()
            reduce_chunk(rows0, EX_PER_CHUNK * i)

            @pl.when(i + 2 < tpw)
            def _():
                pltpu.async_copy(emb_hbm.at[idx_v.at[i + 2]], rows0, sem0)

            pltpu.make_async_copy(emb_hbm.at[idx_v.at[i + 1]], rows1, sem1).wait()
            reduce_chunk(rows1, EX_PER_CHUNK * (i + 1))

            @pl.when(i + 3 < tpw)
            def _():
                pltpu.async_copy(emb_hbm.at[idx_v.at[i + 3]], rows1, sem1)

        pltpu.sync_copy(out_v, out_hbm.at[wid])

    return pool


def _linear(e0, e1, l0, l1, w0, w1, bias):
    B = e0.shape[0]
    labels = w0.shape[0]
    bm = 512
    grid = (B // bm,)

    def body(e0_ref, e1_ref, l0_ref, l1_ref, w0_ref, w1_ref, b_ref, out_ref):
        s0 = e0_ref[...] / l0_ref[...]
        s1 = e1_ref[...] / l1_ref[...]
        dn = (((1,), (1,)), ((), ()))
        p = lax.dot_general(s0, w0_ref[...], dn, preferred_element_type=jnp.float32)
        p = p + lax.dot_general(s1, w1_ref[...], dn, preferred_element_type=jnp.float32)
        out_ref[...] = p + b_ref[...]

    return pl.pallas_call(
        body,
        grid=grid,
        in_specs=[
            pl.BlockSpec((bm, DIM), lambda m: (m, 0)),
            pl.BlockSpec((bm, DIM), lambda m: (m, 0)),
            pl.BlockSpec((bm, 1), lambda m: (m, 0)),
            pl.BlockSpec((bm, 1), lambda m: (m, 0)),
            pl.BlockSpec((labels, DIM), lambda m: (0, 0)),
            pl.BlockSpec((labels, DIM), lambda m: (0, 0)),
            pl.BlockSpec((1, labels), lambda m: (0, 0)),
        ],
        out_specs=pl.BlockSpec((bm, labels), lambda m: (m, 0)),
        out_shape=jax.ShapeDtypeStruct((B, labels), jnp.float32),
    )(e0, e1, l0, l1, w0, w1, bias)


def kernel(x0, x1, length_0, length_1, embedding, W, b):
    B, seq = x0.shape
    assert seq == L and embedding.shape[1] == DIM
    x = jnp.concatenate([x0, x1], axis=0).astype(jnp.int32)
    idx2 = x.reshape(-1, CHUNK)
    total_chunks = idx2.shape[0]

    pooled = _make_pool(total_chunks)(idx2, embedding)
    half = NW // 2
    e0 = pooled[:half].reshape(B, DIM)
    e1 = pooled[half:].reshape(B, DIM)

    pred = _linear(
        e0, e1,
        length_0.reshape(B, 1), length_1.reshape(B, 1),
        W[:, :DIM], W[:, DIM:],
        b.reshape(1, -1),
    )
    return (pred, 0.0)


# speedup vs baseline: 1.5392x; 1.0485x over previous
import functools

import jax
import jax.numpy as jnp
from jax import lax
from jax.experimental import pallas as pl
from jax.experimental.pallas import tpu as pltpu
from jax.experimental.pallas import tpu_sc as plsc

DIM = 64
L = 50
EX_PER_CHUNK = 2
CHUNK = EX_PER_CHUNK * L
NC, NS = 2, 16
NW = NC * NS
NVREG = DIM // 16


ROW = 128


def _make_pool(total_chunks: int):
    tpw = total_chunks // NW
    epw = tpw * EX_PER_CHUNK

    mesh = plsc.VectorSubcoreMesh(core_axis_name="c", subcore_axis_name="s")

    @functools.partial(
        pl.kernel,
        out_type=jax.ShapeDtypeStruct((NW, epw * DIM), jnp.float32),
        mesh=mesh,
        scratch_types=[
            pltpu.VMEM((tpw, CHUNK), jnp.int32),
            pltpu.VMEM((CHUNK, ROW), jnp.float32),
            pltpu.VMEM((CHUNK, ROW), jnp.float32),
            pltpu.VMEM((epw * DIM,), jnp.float32),
            pltpu.SemaphoreType.DMA,
            pltpu.SemaphoreType.DMA,
        ],
    )
    def pool(idx_hbm, emb_hbm, out_hbm, idx_v, rows0, rows1, out_v, sem0, sem1):
        wid = lax.axis_index("c") * NS + lax.axis_index("s")
        pltpu.sync_copy(idx_hbm.at[pl.ds(wid * tpw, tpw)], idx_v)

        def reduce_chunk(rows, local_base):
            for e in range(EX_PER_CHUNK):
                acc = [rows[e * L, pl.ds(c * 16, 16)] for c in range(NVREG)]
                for r in range(1, L):
                    for c in range(NVREG):
                        acc[c] = acc[c] + rows[e * L + r, pl.ds(c * 16, 16)]
                off = (local_base + e) * DIM
                for c in range(NVREG):
                    out_v[pl.ds(off + c * 16, 16)] = acc[c]

        pltpu.async_copy(emb_hbm.at[idx_v.at[0]], rows0, sem0)
        pltpu.async_copy(emb_hbm.at[idx_v.at[1]], rows1, sem1)

        @pl.loop(0, tpw, step=2)
        def _(i):
            pltpu.make_async_copy(emb_hbm.at[idx_v.at[i]], rows0, sem0).wait()
            reduce_chunk(rows0, EX_PER_CHUNK * i)

            @pl.when(i + 2 < tpw)
            def _():
                pltpu.async_copy(emb_hbm.at[idx_v.at[i + 2]], rows0, sem0)

            pltpu.make_async_copy(emb_hbm.at[idx_v.at[i + 1]], rows1, sem1).wait()
            reduce_chunk(rows1, EX_PER_CHUNK * (i + 1))

            @pl.when(i + 3 < tpw)
            def _():
                pltpu.async_copy(emb_hbm.at[idx_v.at[i + 3]], rows1, sem1)

        pltpu.sync_copy(out_v, out_hbm.at[wid])

    return pool


def _linear(e0, e1, l0, l1, w0, w1, bias):
    B = e0.shape[0]
    labels = w0.shape[0]
    bm = 512
    grid = (B // bm,)

    def body(e0_ref, e1_ref, l0_ref, l1_ref, w0_ref, w1_ref, b_ref, out_ref):
        s0 = e0_ref[...] / l0_ref[...]
        s1 = e1_ref[...] / l1_ref[...]
        dn = (((1,), (1,)), ((), ()))
        p = lax.dot_general(s0, w0_ref[...], dn, preferred_element_type=jnp.float32)
        p = p + lax.dot_general(s1, w1_ref[...], dn, preferred_element_type=jnp.float32)
        out_ref[...] = p + b_ref[...]

    return pl.pallas_call(
        body,
        grid=grid,
        in_specs=[
            pl.BlockSpec((bm, DIM), lambda m: (m, 0)),
            pl.BlockSpec((bm, DIM), lambda m: (m, 0)),
            pl.BlockSpec((bm, 1), lambda m: (m, 0)),
            pl.BlockSpec((bm, 1), lambda m: (m, 0)),
            pl.BlockSpec((labels, DIM), lambda m: (0, 0)),
            pl.BlockSpec((labels, DIM), lambda m: (0, 0)),
            pl.BlockSpec((1, labels), lambda m: (0, 0)),
        ],
        out_specs=pl.BlockSpec((bm, labels), lambda m: (m, 0)),
        out_shape=jax.ShapeDtypeStruct((B, labels), jnp.float32),
    )(e0, e1, l0, l1, w0, w1, bias)


def kernel(x0, x1, length_0, length_1, embedding, W, b):
    B, seq = x0.shape
    assert seq == L and embedding.shape[1] == DIM
    x = jnp.concatenate([x0, x1], axis=0).astype(jnp.int32)
    idx2 = x.reshape(-1, CHUNK)
    total_chunks = idx2.shape[0]

    embp = jnp.pad(embedding, ((0, 0), (0, ROW - DIM)))
    pooled = _make_pool(total_chunks)(idx2, embp)
    half = NW // 2
    e0 = pooled[:half].reshape(B, DIM)
    e1 = pooled[half:].reshape(B, DIM)

    pred = _linear(
        e0, e1,
        length_0.reshape(B, 1), length_1.reshape(B, 1),
        W[:, :DIM], W[:, DIM:],
        b.reshape(1, -1),
    )
    return (pred, 0.0)
